# Initial kernel scaffold; baseline (speedup 1.0000x reference)
import jax
import jax.numpy as jnp
from jax.experimental import pallas as pl


def kernel(x, z, edge_index, edge_weight, edge_attr, W_lin1, W_f1, b_f1, W_f2, b_f2, Wq, bq, Wk, bk, Wv, bv, Wo1, bo1, Wo2, bo2):
    raise NotImplementedError("write your pallas kernel here")


# speedup vs baseline: 3.6491x; 3.6491x over previous
import functools
import numpy as np
import jax
import jax.numpy as jnp
from jax import lax
from jax.experimental import pallas as pl
from jax.experimental.pallas import tpu as pltpu
from jax.experimental.pallas import tpu_sc as plsc

N_NODES, N_EDGES, NF, NRBF, NHEADS = 512, 16384, 128, 50, 8
CUTOFF_UPPER = 5.0
NZ = 5
NSEG = NZ * N_NODES
KPAD = 64

NC, NS, L = 2, 16, 16
NW = NC * NS
E_PER_W = N_EDGES // NW
CHUNK = 128
N_CHUNKS = E_PER_W // CHUNK
ROWS_PER_TILE = NSEG // NS

EDGE_TILE = 2048
N_EDGE_TILES = N_EDGES // EDGE_TILE


def _edge_body(ea_ref, w1t_ref, ew_ref, w2t_ref, b2_ref, x_ref, wlt_ref,
               wf_ref, xt_ref):
    i = pl.program_id(0)
    h = jnp.dot(ea_ref[...], w1t_ref[...], preferred_element_type=jnp.float32)
    h = h * jax.nn.sigmoid(h)
    wf = jnp.dot(h, w2t_ref[...], preferred_element_type=jnp.float32)
    wf = wf + b2_ref[...]
    ew = ew_ref[...]
    c = 0.5 * (jnp.cos(ew * (np.pi / CUTOFF_UPPER)) + 1.0)
    c = c * (ew < CUTOFF_UPPER).astype(jnp.float32)
    wf_ref[...] = wf * c

    @pl.when(i == 0)
    def _():
        xt_ref[...] = jnp.dot(x_ref[...], wlt_ref[...],
                              preferred_element_type=jnp.float32)


def _edge_stage(ea_p, w1t_p, ew2d, w2t, b2, x, wlt):
    return pl.pallas_call(
        _edge_body,
        grid=(N_EDGE_TILES,),
        in_specs=[
            pl.BlockSpec((EDGE_TILE, KPAD), lambda i: (i, 0)),
            pl.BlockSpec((KPAD, NF), lambda i: (0, 0)),
            pl.BlockSpec((EDGE_TILE, 1), lambda i: (i, 0)),
            pl.BlockSpec((NF, NF), lambda i: (0, 0)),
            pl.BlockSpec((1, NF), lambda i: (0, 0)),
            pl.BlockSpec((N_NODES, NF), lambda i: (0, 0)),
            pl.BlockSpec((NF, NF), lambda i: (0, 0)),
        ],
        out_specs=[
            pl.BlockSpec((EDGE_TILE, NF), lambda i: (i, 0)),
            pl.BlockSpec((N_NODES, NF), lambda i: (0, 0)),
        ],
        out_shape=[
            jax.ShapeDtypeStruct((N_EDGES, NF), jnp.float32),
            jax.ShapeDtypeStruct((N_NODES, NF), jnp.float32),
        ],
    )(ea_p, w1t_p, ew2d, w2t, b2, x, wlt)


def _sc_body(src_hbm, dst_hbm, z_hbm, xt_hbm, wf_hbm, y_hbm,
             src_v, dst_v, seg_v, rows_v, wf_v, zg_v, y_sh, sem, zsem):
    c = lax.axis_index("c")
    s = lax.axis_index("s")
    wid = c * NS + s

    zero16 = jnp.zeros((L,), jnp.float32)

    def zrow(i, carry):
        for j in range(NF // L):
            rows_v[i, pl.ds(j * L, L)] = zero16
        return carry

    lax.fori_loop(0, CHUNK, zrow, 0)
    pltpu.sync_copy(rows_v, y_sh.at[pl.ds(s * ROWS_PER_TILE, CHUNK)])
    pltpu.sync_copy(rows_v.at[pl.ds(0, ROWS_PER_TILE - CHUNK)],
                    y_sh.at[pl.ds(s * ROWS_PER_TILE + CHUNK,
                                  ROWS_PER_TILE - CHUNK)])
    plsc.subcore_barrier()

    for kc in range(N_CHUNKS):
        base = wid * E_PER_W + kc * CHUNK
        pltpu.sync_copy(src_hbm.at[pl.ds(base, CHUNK)], src_v)
        pltpu.sync_copy(dst_hbm.at[pl.ds(base, CHUNK)], dst_v)
        gat = pltpu.async_copy(xt_hbm.at[src_v], rows_v, sem)
        zgat = pltpu.async_copy(z_hbm.at[src_v], zg_v, zsem)
        pltpu.sync_copy(wf_hbm.at[pl.ds(base, CHUNK)], wf_v)
        zgat.wait()

        def segbody(i, carry):
            sl = pl.ds(i * L, L)
            seg_v[sl] = zg_v[sl] * N_NODES + dst_v[sl]
            return carry

        lax.fori_loop(0, CHUNK // L, segbody, 0)
        gat.wait()

        def mulbody(i, carry):
            for j in range(NF // L):
                sl = pl.ds(j * L, L)
                rows_v[i, sl] = rows_v[i, sl] * wf_v[i, sl]
            return carry

        lax.fori_loop(0, CHUNK, mulbody, 0)
        pltpu.sync_copy(rows_v, y_sh.at[seg_v], add=True)

    plsc.subcore_barrier()
    pltpu.sync_copy(y_sh.at[pl.ds(s * ROWS_PER_TILE, ROWS_PER_TILE)],
                    y_hbm.at[c, pl.ds(s * ROWS_PER_TILE, ROWS_PER_TILE)])


@functools.cache
def _sc_stage_fn():
    return functools.partial(
        pl.kernel,
        out_type=jax.ShapeDtypeStruct((NC, NSEG, NF), jnp.float32),
        mesh=plsc.VectorSubcoreMesh(core_axis_name="c", subcore_axis_name="s",
                                    num_cores=NC, num_subcores=NS),
        scratch_types=[
            pltpu.VMEM((CHUNK,), jnp.int32),
            pltpu.VMEM((CHUNK,), jnp.int32),
            pltpu.VMEM((CHUNK,), jnp.int32),
            pltpu.VMEM((CHUNK, NF), jnp.float32),
            pltpu.VMEM((CHUNK, NF), jnp.float32),
            pltpu.VMEM((CHUNK,), jnp.int32),
            pltpu.VMEM_SHARED((NSEG, NF), jnp.float32),
            pltpu.SemaphoreType.DMA,
            pltpu.SemaphoreType.DMA,
        ],
    )(_sc_body)


def _node_body(yp_ref, wqt_ref, wkt_ref, wvt_ref, wo1t_ref, wo2t_ref, out_ref):
    y = yp_ref[0] + yp_ref[1]
    q = jnp.dot(y, wqt_ref[...], preferred_element_type=jnp.float32)
    k = jnp.dot(y, wkt_ref[...], preferred_element_type=jnp.float32)
    v = jnp.dot(y, wvt_ref[...], preferred_element_type=jnp.float32)
    dh = NF // NHEADS
    lane = lax.broadcasted_iota(jnp.int32, (NF, NHEADS), 0)
    head = lax.broadcasted_iota(jnp.int32, (NF, NHEADS), 1)
    em = (lane // dh == head).astype(jnp.float32)
    lane_t = lax.broadcasted_iota(jnp.int32, (NHEADS, NF), 1)
    head_t = lax.broadcasted_iota(jnp.int32, (NHEADS, NF), 0)
    em_t = (lane_t // dh == head_t).astype(jnp.float32)
    acc = jnp.zeros((N_NODES, NF), jnp.float32)
    for zq in range(NZ):
        qz = lax.slice(q, (zq * N_NODES, 0), ((zq + 1) * N_NODES, NF))
        for zk in range(NZ):
            ky = lax.slice(k, (zk * N_NODES, 0), ((zk + 1) * N_NODES, NF))
            vy = lax.slice(v, (zk * N_NODES, 0), ((zk + 1) * N_NODES, NF))
            sc = jnp.dot(qz * ky, em, preferred_element_type=jnp.float32)
            sc = sc * jax.nn.sigmoid(sc)
            a = jnp.dot(sc, em_t, preferred_element_type=jnp.float32)
            acc = acc + a * vy
    o = jnp.dot(acc, wo1t_ref[...], preferred_element_type=jnp.float32)
    o = jnp.dot(o, wo2t_ref[...], preferred_element_type=jnp.float32)
    out_ref[...] = o * jax.nn.sigmoid(o)


def _node_stage(yp, wqt, wkt, wvt, wo1t, wo2t):
    return pl.pallas_call(
        _node_body,
        out_shape=jax.ShapeDtypeStruct((N_NODES, NF), jnp.float32),
    )(yp, wqt, wkt, wvt, wo1t, wo2t)


def kernel(x, z, edge_index, edge_weight, edge_attr,
           W_lin1, W_f1, b_f1, W_f2, b_f2,
           Wq, bq, Wk, bk, Wv, bv, Wo1, bo1, Wo2, bo2):
    f32 = jnp.float32
    ea_p = jnp.zeros((N_EDGES, KPAD), f32).at[:, :NRBF].set(edge_attr)
    ea_p = ea_p.at[:, NRBF].set(1.0)
    w1t_p = jnp.zeros((KPAD, NF), f32).at[:NRBF, :].set(W_f1.T)
    w1t_p = w1t_p.at[NRBF, :].set(b_f1)
    ew2d = edge_weight.reshape(N_EDGES, 1).astype(f32)
    w2t = W_f2.T.astype(f32)
    wlt = W_lin1.T.astype(f32)

    wf, xt = _edge_stage(ea_p, w1t_p, ew2d, w2t, b_f2.reshape(1, NF),
                         x.astype(f32), wlt)

    src = edge_index[0].astype(jnp.int32)
    dst = edge_index[1].astype(jnp.int32)
    yp = _sc_stage_fn()(src, dst, z.astype(jnp.int32), xt, wf)

    out = _node_stage(yp, Wq.T.astype(f32), Wk.T.astype(f32),
                      Wv.T.astype(f32), Wo1.T.astype(f32), Wo2.T.astype(f32))
    return out


# speedup vs baseline: 5.0122x; 1.3736x over previous
import functools
import numpy as np
import jax
import jax.numpy as jnp
from jax import lax
from jax.experimental import pallas as pl
from jax.experimental.pallas import tpu as pltpu
from jax.experimental.pallas import tpu_sc as plsc

N_NODES, N_EDGES, NF, NRBF, NHEADS = 512, 16384, 128, 50, 8
CUTOFF_UPPER = 5.0
NZ = 5
NSEG = NZ * N_NODES
KPAD = 64

NC, NS, L = 2, 16, 16
NW = NC * NS
E_PER_W = N_EDGES // NW
CHUNK = 128
N_CHUNKS = E_PER_W // CHUNK
ROWS_PER_TILE = NSEG // NS

EDGE_TILE = 2048
N_EDGE_TILES = N_EDGES // EDGE_TILE


def _edge_body(ea_ref, w1t_ref, b1_ref, ew_ref, w2t_ref, b2_ref, x_ref,
               wlt_ref, wf_ref, xt_ref):
    i = pl.program_id(0)
    h = jnp.dot(ea_ref[...], w1t_ref[...], preferred_element_type=jnp.float32)
    h = h + b1_ref[...]
    h = h * jax.nn.sigmoid(h)
    wf = jnp.dot(h, w2t_ref[...], preferred_element_type=jnp.float32)
    wf = wf + b2_ref[...]
    ew = ew_ref[...]
    c = 0.5 * (jnp.cos(ew * (np.pi / CUTOFF_UPPER)) + 1.0)
    c = c * (ew < CUTOFF_UPPER).astype(jnp.float32)
    wf_ref[...] = wf * c

    @pl.when(i == 0)
    def _():
        xt_ref[...] = jnp.dot(x_ref[...], wlt_ref[...],
                              preferred_element_type=jnp.float32)


def _edge_stage(ea, w1t, b1, ew2d, w2t, b2, x, wlt):
    return pl.pallas_call(
        _edge_body,
        grid=(N_EDGE_TILES,),
        in_specs=[
            pl.BlockSpec((EDGE_TILE, NRBF), lambda i: (i, 0)),
            pl.BlockSpec((NRBF, NF), lambda i: (0, 0)),
            pl.BlockSpec((1, NF), lambda i: (0, 0)),
            pl.BlockSpec((EDGE_TILE, 1), lambda i: (i, 0)),
            pl.BlockSpec((NF, NF), lambda i: (0, 0)),
            pl.BlockSpec((1, NF), lambda i: (0, 0)),
            pl.BlockSpec((N_NODES, NF), lambda i: (0, 0)),
            pl.BlockSpec((NF, NF), lambda i: (0, 0)),
        ],
        out_specs=[
            pl.BlockSpec((EDGE_TILE, NF), lambda i: (i, 0)),
            pl.BlockSpec((N_NODES, NF), lambda i: (0, 0)),
        ],
        out_shape=[
            jax.ShapeDtypeStruct((N_EDGES, NF), jnp.float32),
            jax.ShapeDtypeStruct((N_NODES, NF), jnp.float32),
        ],
    )(ea, w1t, b1, ew2d, w2t, b2, x, wlt)


def _sc_body(ei_hbm, z_hbm, xt_hbm, wf_hbm, y_hbm,
             src_v, dst_v, seg_v, rows_v, wf_v, zg_v, y_sh, sem, zsem):
    c = lax.axis_index("c")
    s = lax.axis_index("s")
    wid = c * NS + s

    zero16 = jnp.zeros((L,), jnp.float32)

    def zrow(i, carry):
        for j in range(NF // L):
            rows_v[i, pl.ds(j * L, L)] = zero16
        return carry

    lax.fori_loop(0, CHUNK, zrow, 0)
    pltpu.sync_copy(rows_v, y_sh.at[pl.ds(s * ROWS_PER_TILE, CHUNK)])
    pltpu.sync_copy(rows_v.at[pl.ds(0, ROWS_PER_TILE - CHUNK)],
                    y_sh.at[pl.ds(s * ROWS_PER_TILE + CHUNK,
                                  ROWS_PER_TILE - CHUNK)])
    plsc.subcore_barrier()

    for kc in range(N_CHUNKS):
        base = wid * E_PER_W + kc * CHUNK
        pltpu.sync_copy(ei_hbm.at[0, pl.ds(base, CHUNK)], src_v)
        pltpu.sync_copy(ei_hbm.at[1, pl.ds(base, CHUNK)], dst_v)
        gat = pltpu.async_copy(xt_hbm.at[src_v], rows_v, sem)
        zgat = pltpu.async_copy(z_hbm.at[src_v], zg_v, zsem)
        pltpu.sync_copy(wf_hbm.at[pl.ds(base, CHUNK)], wf_v)
        zgat.wait()

        def segbody(i, carry):
            sl = pl.ds(i * L, L)
            seg_v[sl] = zg_v[sl] * N_NODES + dst_v[sl]
            return carry

        lax.fori_loop(0, CHUNK // L, segbody, 0)
        gat.wait()

        def mulbody(i, carry):
            for j in range(NF // L):
                sl = pl.ds(j * L, L)
                rows_v[i, sl] = rows_v[i, sl] * wf_v[i, sl]
            return carry

        lax.fori_loop(0, CHUNK, mulbody, 0)
        pltpu.sync_copy(rows_v, y_sh.at[seg_v], add=True)

    plsc.subcore_barrier()
    pltpu.sync_copy(y_sh.at[pl.ds(s * ROWS_PER_TILE, ROWS_PER_TILE)],
                    y_hbm.at[c, pl.ds(s * ROWS_PER_TILE, ROWS_PER_TILE)])


@functools.cache
def _sc_stage_fn():
    return functools.partial(
        pl.kernel,
        out_type=jax.ShapeDtypeStruct((NC, NSEG, NF), jnp.float32),
        mesh=plsc.VectorSubcoreMesh(core_axis_name="c", subcore_axis_name="s",
                                    num_cores=NC, num_subcores=NS),
        scratch_types=[
            pltpu.VMEM((CHUNK,), jnp.int32),
            pltpu.VMEM((CHUNK,), jnp.int32),
            pltpu.VMEM((CHUNK,), jnp.int32),
            pltpu.VMEM((CHUNK, NF), jnp.float32),
            pltpu.VMEM((CHUNK, NF), jnp.float32),
            pltpu.VMEM((CHUNK,), jnp.int32),
            pltpu.VMEM_SHARED((NSEG, NF), jnp.float32),
            pltpu.SemaphoreType.DMA,
            pltpu.SemaphoreType.DMA,
        ],
    )(_sc_body)


def _node_body(yp_ref, wqt_ref, wkt_ref, wvt_ref, wo1t_ref, wo2t_ref, out_ref):
    y = yp_ref[0] + yp_ref[1]
    q = jnp.dot(y, wqt_ref[...], preferred_element_type=jnp.float32)
    k = jnp.dot(y, wkt_ref[...], preferred_element_type=jnp.float32)
    v = jnp.dot(y, wvt_ref[...], preferred_element_type=jnp.float32)
    dh = NF // NHEADS
    lane = lax.broadcasted_iota(jnp.int32, (NF, NHEADS), 0)
    head = lax.broadcasted_iota(jnp.int32, (NF, NHEADS), 1)
    em = (lane // dh == head).astype(jnp.float32)
    lane_t = lax.broadcasted_iota(jnp.int32, (NHEADS, NF), 1)
    head_t = lax.broadcasted_iota(jnp.int32, (NHEADS, NF), 0)
    em_t = (lane_t // dh == head_t).astype(jnp.float32)
    acc = jnp.zeros((N_NODES, NF), jnp.float32)
    for zq in range(NZ):
        qz = lax.slice(q, (zq * N_NODES, 0), ((zq + 1) * N_NODES, NF))
        for zk in range(NZ):
            ky = lax.slice(k, (zk * N_NODES, 0), ((zk + 1) * N_NODES, NF))
            vy = lax.slice(v, (zk * N_NODES, 0), ((zk + 1) * N_NODES, NF))
            sc = jnp.dot(qz * ky, em, preferred_element_type=jnp.float32)
            sc = sc * jax.nn.sigmoid(sc)
            a = jnp.dot(sc, em_t, preferred_element_type=jnp.float32)
            acc = acc + a * vy
    o = jnp.dot(acc, wo1t_ref[...], preferred_element_type=jnp.float32)
    o = jnp.dot(o, wo2t_ref[...], preferred_element_type=jnp.float32)
    out_ref[...] = o * jax.nn.sigmoid(o)


def _node_stage(yp, wqt, wkt, wvt, wo1t, wo2t):
    return pl.pallas_call(
        _node_body,
        out_shape=jax.ShapeDtypeStruct((N_NODES, NF), jnp.float32),
    )(yp, wqt, wkt, wvt, wo1t, wo2t)


def kernel(x, z, edge_index, edge_weight, edge_attr,
           W_lin1, W_f1, b_f1, W_f2, b_f2,
           Wq, bq, Wk, bk, Wv, bv, Wo1, bo1, Wo2, bo2):
    f32 = jnp.float32
    ew2d = edge_weight.reshape(N_EDGES, 1).astype(f32)

    wf, xt = _edge_stage(edge_attr.astype(f32), W_f1.T.astype(f32),
                         b_f1.reshape(1, NF), ew2d, W_f2.T.astype(f32),
                         b_f2.reshape(1, NF), x.astype(f32),
                         W_lin1.T.astype(f32))

    yp = _sc_stage_fn()(edge_index.astype(jnp.int32), z.astype(jnp.int32),
                        xt, wf)

    out = _node_stage(yp, Wq.T.astype(f32), Wk.T.astype(f32),
                      Wv.T.astype(f32), Wo1.T.astype(f32), Wo2.T.astype(f32))
    return out


# speedup vs baseline: 5.0785x; 1.0132x over previous
import functools
import numpy as np
import jax
import jax.numpy as jnp
from jax import lax
from jax.experimental import pallas as pl
from jax.experimental.pallas import tpu as pltpu
from jax.experimental.pallas import tpu_sc as plsc

N_NODES, N_EDGES, NF, NRBF, NHEADS = 512, 16384, 128, 50, 8
CUTOFF_UPPER = 5.0
NZ = 5
NSEG = NZ * N_NODES
KPAD = 64

NC, NS, L = 2, 16, 16
NW = NC * NS
E_PER_W = N_EDGES // NW
CHUNK = 128
N_CHUNKS = E_PER_W // CHUNK
ROWS_PER_TILE = NSEG // NS

EDGE_TILE = 2048
N_EDGE_TILES = N_EDGES // EDGE_TILE


def _edge_body(ea_ref, w1t_ref, b1_ref, ew_ref, w2t_ref, b2_ref, x_ref,
               wlt_ref, wf_ref, xt_ref):
    i = pl.program_id(0)
    h = jnp.dot(ea_ref[...], w1t_ref[...], preferred_element_type=jnp.float32)
    h = h + b1_ref[...]
    h = h * jax.nn.sigmoid(h)
    wf = jnp.dot(h, w2t_ref[...], preferred_element_type=jnp.float32)
    wf = wf + b2_ref[...]
    ew = ew_ref[...]
    c = 0.5 * (jnp.cos(ew * (np.pi / CUTOFF_UPPER)) + 1.0)
    c = c * (ew < CUTOFF_UPPER).astype(jnp.float32)
    wf_ref[...] = wf * c

    @pl.when(i == 0)
    def _():
        xt_ref[...] = jnp.dot(x_ref[...], wlt_ref[...],
                              preferred_element_type=jnp.float32)


def _edge_stage(ea, w1t, b1, ew2d, w2t, b2, x, wlt):
    return pl.pallas_call(
        _edge_body,
        grid=(N_EDGE_TILES,),
        in_specs=[
            pl.BlockSpec((EDGE_TILE, NRBF), lambda i: (i, 0)),
            pl.BlockSpec((NRBF, NF), lambda i: (0, 0)),
            pl.BlockSpec((1, NF), lambda i: (0, 0)),
            pl.BlockSpec((EDGE_TILE, 1), lambda i: (i, 0)),
            pl.BlockSpec((NF, NF), lambda i: (0, 0)),
            pl.BlockSpec((1, NF), lambda i: (0, 0)),
            pl.BlockSpec((N_NODES, NF), lambda i: (0, 0)),
            pl.BlockSpec((NF, NF), lambda i: (0, 0)),
        ],
        out_specs=[
            pl.BlockSpec((EDGE_TILE, NF), lambda i: (i, 0)),
            pl.BlockSpec((N_NODES, NF), lambda i: (0, 0)),
        ],
        out_shape=[
            jax.ShapeDtypeStruct((N_EDGES, NF), jnp.float32),
            jax.ShapeDtypeStruct((N_NODES, NF), jnp.float32),
        ],
    )(ea, w1t, b1, ew2d, w2t, b2, x, wlt)


def _sc_body(ei_hbm, z_hbm, xt_hbm, wf_hbm, y_hbm,
             src0, src1, src2, src3, dst0, dst1, dst2, dst3,
             seg0, seg1, rows0, rows1, rows2, wf0, wf1, zg0, zg1, zbuf_v,
             y_sh, gsem0, gsem1, gsem2, wsem0, wsem1, zsem0, zsem1):
    c = lax.axis_index("c")
    s = lax.axis_index("s")
    wid = c * NS + s
    base0 = wid * E_PER_W
    srcs = [src0, src1, src2, src3]
    dsts = [dst0, dst1, dst2, dst3]
    segs = [seg0, seg1]
    rows = [rows0, rows1, rows2]
    wfs = [wf0, wf1]
    zgs = [zg0, zg1]
    gsems = [gsem0, gsem1, gsem2]
    wsems = [wsem0, wsem1]
    zsems = [zsem0, zsem1]

    for kc in range(N_CHUNKS):
        pltpu.sync_copy(ei_hbm.at[0, pl.ds(base0 + kc * CHUNK, CHUNK)],
                        srcs[kc])
        pltpu.sync_copy(ei_hbm.at[1, pl.ds(base0 + kc * CHUNK, CHUNK)],
                        dsts[kc])

    gds, zds, wds = {}, {}, {}

    def start_fetch(kc):
        gds[kc] = pltpu.make_async_copy(xt_hbm.at[srcs[kc]], rows[kc % 3],
                                        gsems[kc % 3])
        gds[kc].start()
        zds[kc] = pltpu.make_async_copy(z_hbm.at[srcs[kc]], zgs[kc % 2],
                                        zsems[kc % 2])
        zds[kc].start()
        wds[kc] = pltpu.make_async_copy(
            wf_hbm.at[pl.ds(base0 + kc * CHUNK, CHUNK)],
            wfs[kc % 2], wsems[kc % 2])
        wds[kc].start()

    start_fetch(0)
    start_fetch(1)

    zero16 = jnp.zeros((L,), jnp.float32)

    def zrow(i, carry):
        for j in range(NF // L):
            zbuf_v[i, pl.ds(j * L, L)] = zero16
        return carry

    lax.fori_loop(0, ROWS_PER_TILE // 5, zrow, 0)
    for r in range(5):
        pltpu.sync_copy(zbuf_v, y_sh.at[pl.ds(s * ROWS_PER_TILE +
                                              r * (ROWS_PER_TILE // 5),
                                              ROWS_PER_TILE // 5)])
    plsc.subcore_barrier()

    for kc in range(N_CHUNKS):
        cur3 = kc % 3
        cur2 = kc % 2
        zds[kc].wait()

        def segbody(i, carry):
            sl = pl.ds(i * L, L)
            segs[cur2][sl] = zgs[cur2][sl] * N_NODES + dsts[kc][sl]
            return carry

        lax.fori_loop(0, CHUNK // L, segbody, 0)
        gds[kc].wait()
        wds[kc].wait()

        def mulbody(i, carry):
            for j in range(NF // L):
                sl = pl.ds(j * L, L)
                rows[cur3][i, sl] = rows[cur3][i, sl] * wfs[cur2][i, sl]
            return carry

        lax.fori_loop(0, CHUNK, mulbody, 0)
        if kc + 2 < N_CHUNKS:
            start_fetch(kc + 2)
        pltpu.sync_copy(rows[cur3], y_sh.at[segs[cur2]], add=True)

    plsc.subcore_barrier()
    pltpu.sync_copy(y_sh.at[pl.ds(s * ROWS_PER_TILE, ROWS_PER_TILE)],
                    y_hbm.at[c, pl.ds(s * ROWS_PER_TILE, ROWS_PER_TILE)])


@functools.cache
def _sc_stage_fn():
    return functools.partial(
        pl.kernel,
        out_type=jax.ShapeDtypeStruct((NC, NSEG, NF), jnp.float32),
        mesh=plsc.VectorSubcoreMesh(core_axis_name="c", subcore_axis_name="s",
                                    num_cores=NC, num_subcores=NS),
        scratch_types=(
            [pltpu.VMEM((CHUNK,), jnp.int32)] * 8 +
            [pltpu.VMEM((CHUNK,), jnp.int32)] * 2 +
            [pltpu.VMEM((CHUNK, NF), jnp.float32)] * 3 +
            [pltpu.VMEM((CHUNK, NF), jnp.float32)] * 2 +
            [pltpu.VMEM((CHUNK,), jnp.int32)] * 2 +
            [pltpu.VMEM((ROWS_PER_TILE // 5, NF), jnp.float32),
             pltpu.VMEM_SHARED((NSEG, NF), jnp.float32)] +
            [pltpu.SemaphoreType.DMA] * 7
        ),
    )(_sc_body)


def _node_body(yp_ref, wqt_ref, wkt_ref, wvt_ref, wo1t_ref, wo2t_ref, out_ref):
    y = yp_ref[0] + yp_ref[1]
    q = jnp.dot(y, wqt_ref[...], preferred_element_type=jnp.float32)
    k = jnp.dot(y, wkt_ref[...], preferred_element_type=jnp.float32)
    v = jnp.dot(y, wvt_ref[...], preferred_element_type=jnp.float32)
    dh = NF // NHEADS
    lane = lax.broadcasted_iota(jnp.int32, (NF, NHEADS), 0)
    head = lax.broadcasted_iota(jnp.int32, (NF, NHEADS), 1)
    em = (lane // dh == head).astype(jnp.float32)
    lane_t = lax.broadcasted_iota(jnp.int32, (NHEADS, NF), 1)
    head_t = lax.broadcasted_iota(jnp.int32, (NHEADS, NF), 0)
    em_t = (lane_t // dh == head_t).astype(jnp.float32)
    acc = jnp.zeros((N_NODES, NF), jnp.float32)
    for zq in range(NZ):
        qz = lax.slice(q, (zq * N_NODES, 0), ((zq + 1) * N_NODES, NF))
        for zk in range(NZ):
            ky = lax.slice(k, (zk * N_NODES, 0), ((zk + 1) * N_NODES, NF))
            vy = lax.slice(v, (zk * N_NODES, 0), ((zk + 1) * N_NODES, NF))
            sc = jnp.dot(qz * ky, em, preferred_element_type=jnp.float32)
            sc = sc * jax.nn.sigmoid(sc)
            a = jnp.dot(sc, em_t, preferred_element_type=jnp.float32)
            acc = acc + a * vy
    o = jnp.dot(acc, wo1t_ref[...], preferred_element_type=jnp.float32)
    o = jnp.dot(o, wo2t_ref[...], preferred_element_type=jnp.float32)
    out_ref[...] = o * jax.nn.sigmoid(o)


def _node_stage(yp, wqt, wkt, wvt, wo1t, wo2t):
    return pl.pallas_call(
        _node_body,
        out_shape=jax.ShapeDtypeStruct((N_NODES, NF), jnp.float32),
    )(yp, wqt, wkt, wvt, wo1t, wo2t)


def kernel(x, z, edge_index, edge_weight, edge_attr,
           W_lin1, W_f1, b_f1, W_f2, b_f2,
           Wq, bq, Wk, bk, Wv, bv, Wo1, bo1, Wo2, bo2):
    f32 = jnp.float32
    ew2d = edge_weight.reshape(N_EDGES, 1).astype(f32)

    wf, xt = _edge_stage(edge_attr.astype(f32), W_f1.T.astype(f32),
                         b_f1.reshape(1, NF), ew2d, W_f2.T.astype(f32),
                         b_f2.reshape(1, NF), x.astype(f32),
                         W_lin1.T.astype(f32))

    yp = _sc_stage_fn()(edge_index.astype(jnp.int32), z.astype(jnp.int32),
                        xt, wf)

    out = _node_stage(yp, Wq.T.astype(f32), Wk.T.astype(f32),
                      Wv.T.astype(f32), Wo1.T.astype(f32), Wo2.T.astype(f32))
    return out


# speedup vs baseline: 5.2537x; 1.0345x over previous
import functools
import numpy as np
import jax
import jax.numpy as jnp
from jax import lax
from jax.experimental import pallas as pl
from jax.experimental.pallas import tpu as pltpu
from jax.experimental.pallas import tpu_sc as plsc

N_NODES, N_EDGES, NF, NRBF, NHEADS = 512, 16384, 128, 50, 8
CUTOFF_UPPER = 5.0
NZ = 5
NSEG = NZ * N_NODES

NC, NS, L = 2, 16, 16
NW = NC * NS
N_HALF = N_EDGES // 2
EH_PER_W = N_HALF // NW
CHUNK = 128
N_CHUNKS_H = EH_PER_W // CHUNK
ROWS_PER_TILE = NSEG // NS

EDGE_TILE = 2048
N_TILES_HALF = N_HALF // EDGE_TILE


def _filter_block(ea, w1t, b1, ew, w2t, b2):
    h = jnp.dot(ea, w1t, preferred_element_type=jnp.float32)
    h = h + b1
    h = h * jax.nn.sigmoid(h)
    wf = jnp.dot(h, w2t, preferred_element_type=jnp.float32)
    wf = wf + b2
    c = 0.5 * (jnp.cos(ew * (np.pi / CUTOFF_UPPER)) + 1.0)
    c = c * (ew < CUTOFF_UPPER).astype(jnp.float32)
    return wf * c


def _edge_body_a(ea_ref, w1t_ref, b1_ref, ew_ref, w2t_ref, b2_ref, x_ref,
                 wlt_ref, wf_ref, xt_ref):
    wf_ref[...] = _filter_block(ea_ref[...], w1t_ref[...], b1_ref[...],
                                ew_ref[...], w2t_ref[...], b2_ref[...])

    @pl.when(pl.program_id(0) == 0)
    def _():
        xt_ref[...] = jnp.dot(x_ref[...], wlt_ref[...],
                              preferred_element_type=jnp.float32)


def _edge_body_b(ea_ref, w1t_ref, b1_ref, ew_ref, w2t_ref, b2_ref, wf_ref):
    wf_ref[...] = _filter_block(ea_ref[...], w1t_ref[...], b1_ref[...],
                                ew_ref[...], w2t_ref[...], b2_ref[...])


def _edge_stage_a(ea, w1t, b1, ew2d, w2t, b2, x, wlt):
    return pl.pallas_call(
        _edge_body_a,
        grid=(N_TILES_HALF,),
        in_specs=[
            pl.BlockSpec((EDGE_TILE, NRBF), lambda i: (i, 0)),
            pl.BlockSpec((NRBF, NF), lambda i: (0, 0)),
            pl.BlockSpec((1, NF), lambda i: (0, 0)),
            pl.BlockSpec((EDGE_TILE, 1), lambda i: (i, 0)),
            pl.BlockSpec((NF, NF), lambda i: (0, 0)),
            pl.BlockSpec((1, NF), lambda i: (0, 0)),
            pl.BlockSpec((N_NODES, NF), lambda i: (0, 0)),
            pl.BlockSpec((NF, NF), lambda i: (0, 0)),
        ],
        out_specs=[
            pl.BlockSpec((EDGE_TILE, NF), lambda i: (i, 0)),
            pl.BlockSpec((N_NODES, NF), lambda i: (0, 0)),
        ],
        out_shape=[
            jax.ShapeDtypeStruct((N_HALF, NF), jnp.float32),
            jax.ShapeDtypeStruct((N_NODES, NF), jnp.float32),
        ],
    )(ea, w1t, b1, ew2d, w2t, b2, x, wlt)


def _edge_stage_b(ea, w1t, b1, ew2d, w2t, b2):
    return pl.pallas_call(
        _edge_body_b,
        grid=(N_TILES_HALF,),
        in_specs=[
            pl.BlockSpec((EDGE_TILE, NRBF), lambda i: (i + N_TILES_HALF, 0)),
            pl.BlockSpec((NRBF, NF), lambda i: (0, 0)),
            pl.BlockSpec((1, NF), lambda i: (0, 0)),
            pl.BlockSpec((EDGE_TILE, 1), lambda i: (i + N_TILES_HALF, 0)),
            pl.BlockSpec((NF, NF), lambda i: (0, 0)),
            pl.BlockSpec((1, NF), lambda i: (0, 0)),
        ],
        out_specs=pl.BlockSpec((EDGE_TILE, NF), lambda i: (i, 0)),
        out_shape=jax.ShapeDtypeStruct((N_HALF, NF), jnp.float32),
    )(ea, w1t, b1, ew2d, w2t, b2)


def _make_sc_body(half):
    half_off = half * N_HALF

    def _sc_body(ei_hbm, z_hbm, xt_hbm, wf_hbm, y_hbm,
                 src0, src1, dst0, dst1, seg0, seg1,
                 rows0, rows1, wf0, wf1, zg0, zg1, zbuf_v, y_sh,
                 gsem0, gsem1, wsem0, wsem1, zsem0, zsem1):
        c = lax.axis_index("c")
        s = lax.axis_index("s")
        wid = c * NS + s
        ebase = half_off + wid * EH_PER_W
        wbase = wid * EH_PER_W
        srcs = [src0, src1]
        dsts = [dst0, dst1]
        segs = [seg0, seg1]
        rows = [rows0, rows1]
        wfs = [wf0, wf1]
        zgs = [zg0, zg1]
        gsems = [gsem0, gsem1]
        wsems = [wsem0, wsem1]
        zsems = [zsem0, zsem1]

        for kc in range(N_CHUNKS_H):
            pltpu.sync_copy(ei_hbm.at[0, pl.ds(ebase + kc * CHUNK, CHUNK)],
                            srcs[kc])
            pltpu.sync_copy(ei_hbm.at[1, pl.ds(ebase + kc * CHUNK, CHUNK)],
                            dsts[kc])

        gds, zds, wds = {}, {}, {}
        for kc in range(N_CHUNKS_H):
            gds[kc] = pltpu.make_async_copy(xt_hbm.at[srcs[kc]], rows[kc],
                                            gsems[kc])
            gds[kc].start()
            zds[kc] = pltpu.make_async_copy(z_hbm.at[srcs[kc]], zgs[kc],
                                            zsems[kc])
            zds[kc].start()
            wds[kc] = pltpu.make_async_copy(
                wf_hbm.at[pl.ds(wbase + kc * CHUNK, CHUNK)], wfs[kc],
                wsems[kc])
            wds[kc].start()

        zero16 = jnp.zeros((L,), jnp.float32)

        def zrow(i, carry):
            for j in range(NF // L):
                zbuf_v[i, pl.ds(j * L, L)] = zero16
            return carry

        lax.fori_loop(0, ROWS_PER_TILE // 5, zrow, 0)
        for r in range(5):
            pltpu.sync_copy(zbuf_v, y_sh.at[pl.ds(s * ROWS_PER_TILE +
                                                  r * (ROWS_PER_TILE // 5),
                                                  ROWS_PER_TILE // 5)])
        plsc.subcore_barrier()

        for kc in range(N_CHUNKS_H):
            zds[kc].wait()

            def segbody(i, carry):
                sl = pl.ds(i * L, L)
                segs[kc][sl] = zgs[kc][sl] * N_NODES + dsts[kc][sl]
                return carry

            lax.fori_loop(0, CHUNK // L, segbody, 0)
            gds[kc].wait()
            wds[kc].wait()

            def mulbody(i, carry):
                for j in range(NF // L):
                    sl = pl.ds(j * L, L)
                    rows[kc][i, sl] = rows[kc][i, sl] * wfs[kc][i, sl]
                return carry

            lax.fori_loop(0, CHUNK, mulbody, 0)
            pltpu.sync_copy(rows[kc], y_sh.at[segs[kc]], add=True)

        plsc.subcore_barrier()
        pltpu.sync_copy(y_sh.at[pl.ds(s * ROWS_PER_TILE, ROWS_PER_TILE)],
                        y_hbm.at[c, pl.ds(s * ROWS_PER_TILE, ROWS_PER_TILE)])

    return _sc_body


@functools.cache
def _sc_stage_fn(half):
    return functools.partial(
        pl.kernel,
        out_type=jax.ShapeDtypeStruct((NC, NSEG, NF), jnp.float32),
        mesh=plsc.VectorSubcoreMesh(core_axis_name="c", subcore_axis_name="s",
                                    num_cores=NC, num_subcores=NS),
        scratch_types=(
            [pltpu.VMEM((CHUNK,), jnp.int32)] * 4 +
            [pltpu.VMEM((CHUNK,), jnp.int32)] * 2 +
            [pltpu.VMEM((CHUNK, NF), jnp.float32)] * 2 +
            [pltpu.VMEM((CHUNK, NF), jnp.float32)] * 2 +
            [pltpu.VMEM((CHUNK,), jnp.int32)] * 2 +
            [pltpu.VMEM((ROWS_PER_TILE // 5, NF), jnp.float32),
             pltpu.VMEM_SHARED((NSEG, NF), jnp.float32)] +
            [pltpu.SemaphoreType.DMA] * 6
        ),
    )(_make_sc_body(half))


def _node_body(yp1_ref, yp2_ref, wqt_ref, wkt_ref, wvt_ref, wo1t_ref,
               wo2t_ref, out_ref):
    y = (yp1_ref[0] + yp1_ref[1]) + (yp2_ref[0] + yp2_ref[1])
    q = jnp.dot(y, wqt_ref[...], preferred_element_type=jnp.float32)
    k = jnp.dot(y, wkt_ref[...], preferred_element_type=jnp.float32)
    v = jnp.dot(y, wvt_ref[...], preferred_element_type=jnp.float32)
    dh = NF // NHEADS
    lane = lax.broadcasted_iota(jnp.int32, (NF, NHEADS), 0)
    head = lax.broadcasted_iota(jnp.int32, (NF, NHEADS), 1)
    em = (lane // dh == head).astype(jnp.float32)
    lane_t = lax.broadcasted_iota(jnp.int32, (NHEADS, NF), 1)
    head_t = lax.broadcasted_iota(jnp.int32, (NHEADS, NF), 0)
    em_t = (lane_t // dh == head_t).astype(jnp.float32)
    acc = jnp.zeros((N_NODES, NF), jnp.float32)
    for zq in range(NZ):
        qz = lax.slice(q, (zq * N_NODES, 0), ((zq + 1) * N_NODES, NF))
        for zk in range(NZ):
            ky = lax.slice(k, (zk * N_NODES, 0), ((zk + 1) * N_NODES, NF))
            vy = lax.slice(v, (zk * N_NODES, 0), ((zk + 1) * N_NODES, NF))
            sc = jnp.dot(qz * ky, em, preferred_element_type=jnp.float32)
            sc = sc * jax.nn.sigmoid(sc)
            a = jnp.dot(sc, em_t, preferred_element_type=jnp.float32)
            acc = acc + a * vy
    o = jnp.dot(acc, wo1t_ref[...], preferred_element_type=jnp.float32)
    o = jnp.dot(o, wo2t_ref[...], preferred_element_type=jnp.float32)
    out_ref[...] = o * jax.nn.sigmoid(o)


def _node_stage(yp1, yp2, wqt, wkt, wvt, wo1t, wo2t):
    return pl.pallas_call(
        _node_body,
        out_shape=jax.ShapeDtypeStruct((N_NODES, NF), jnp.float32),
    )(yp1, yp2, wqt, wkt, wvt, wo1t, wo2t)


def kernel(x, z, edge_index, edge_weight, edge_attr,
           W_lin1, W_f1, b_f1, W_f2, b_f2,
           Wq, bq, Wk, bk, Wv, bv, Wo1, bo1, Wo2, bo2):
    f32 = jnp.float32
    ew2d = edge_weight.reshape(N_EDGES, 1).astype(f32)
    ea = edge_attr.astype(f32)
    w1t = W_f1.T.astype(f32)
    b1 = b_f1.reshape(1, NF)
    w2t = W_f2.T.astype(f32)
    b2 = b_f2.reshape(1, NF)
    ei = edge_index.astype(jnp.int32)
    zi = z.astype(jnp.int32)

    wf_lo, xt = _edge_stage_a(ea, w1t, b1, ew2d, w2t, b2, x.astype(f32),
                              W_lin1.T.astype(f32))
    yp1 = _sc_stage_fn(0)(ei, zi, xt, wf_lo)
    wf_hi = _edge_stage_b(ea, w1t, b1, ew2d, w2t, b2)
    yp2 = _sc_stage_fn(1)(ei, zi, xt, wf_hi)

    out = _node_stage(yp1, yp2, Wq.T.astype(f32), Wk.T.astype(f32),
                      Wv.T.astype(f32), Wo1.T.astype(f32), Wo2.T.astype(f32))
    return out


# speedup vs baseline: 7.0230x; 1.3368x over previous
import functools
import numpy as np
import jax
import jax.numpy as jnp
from jax import lax
from jax.experimental import pallas as pl
from jax.experimental.pallas import tpu as pltpu
from jax.experimental.pallas import tpu_sc as plsc

N_NODES, N_EDGES, NF, NRBF, NHEADS = 512, 16384, 128, 50, 8
CUTOFF_UPPER = 5.0
NZ = 5
NSEG = NZ * N_NODES

NC, NS, L = 2, 16, 16
NW = NC * NS
N_HALF = N_EDGES // 2
EH_PER_W = N_HALF // NW
CHUNK = 128
N_CHUNKS_H = EH_PER_W // CHUNK
ROWS_PER_TILE = NSEG // NS

EDGE_TILE = 2048
N_TILES_HALF = N_HALF // EDGE_TILE


def _filter_block(ea_t, w1, b1, ew2, w2, b2):
    h = lax.dot_general(ea_t, w1, (((0,), (1,)), ((), ())),
                        preferred_element_type=jnp.float32)
    h = h + b1
    h = h * jax.nn.sigmoid(h)
    wf = lax.dot_general(h, w2, (((1,), (1,)), ((), ())),
                         preferred_element_type=jnp.float32)
    wf = wf + b2
    c = 0.5 * (jnp.cos(ew2 * (np.pi / CUTOFF_UPPER)) + 1.0)
    c = c * (ew2 < CUTOFF_UPPER).astype(jnp.float32)
    ct = c.T
    scaled = []
    for j in range(ct.shape[1]):
        col = lax.slice(ct, (0, j), (128, j + 1))
        slab = lax.slice(wf, (j * 128, 0), ((j + 1) * 128, NF))
        scaled.append(slab * col)
    return jnp.concatenate(scaled, axis=0)


def _edge_body_a(ea_ref, w1t_ref, b1_ref, ew_ref, w2t_ref, b2_ref, x_ref,
                 wlt_ref, wf_ref, xt_ref):
    wf_ref[...] = _filter_block(ea_ref[...], w1t_ref[...], b1_ref[...],
                                ew_ref[...], w2t_ref[...], b2_ref[...])

    @pl.when(pl.program_id(0) == 0)
    def _():
        xt_ref[...] = lax.dot_general(x_ref[...], wlt_ref[...],
                                      (((1,), (1,)), ((), ())),
                                      preferred_element_type=jnp.float32)


def _edge_body_b(ea_ref, w1t_ref, b1_ref, ew_ref, w2t_ref, b2_ref, wf_ref):
    wf_ref[...] = _filter_block(ea_ref[...], w1t_ref[...], b1_ref[...],
                                ew_ref[...], w2t_ref[...], b2_ref[...])


def _edge_stage_a(ea, w1t, b1, ew2d, w2t, b2, x, wlt):
    return pl.pallas_call(
        _edge_body_a,
        grid=(N_TILES_HALF,),
        in_specs=[
            pl.BlockSpec((NRBF, EDGE_TILE), lambda i: (0, i)),
            pl.BlockSpec((NF, NRBF), lambda i: (0, 0)),
            pl.BlockSpec((1, NF), lambda i: (0, 0)),
            pl.BlockSpec((EDGE_TILE // 128, 128), lambda i: (i, 0)),
            pl.BlockSpec((NF, NF), lambda i: (0, 0)),
            pl.BlockSpec((1, NF), lambda i: (0, 0)),
            pl.BlockSpec((N_NODES, NF), lambda i: (0, 0)),
            pl.BlockSpec((NF, NF), lambda i: (0, 0)),
        ],
        out_specs=[
            pl.BlockSpec((EDGE_TILE, NF), lambda i: (i, 0)),
            pl.BlockSpec((N_NODES, NF), lambda i: (0, 0)),
        ],
        out_shape=[
            jax.ShapeDtypeStruct((N_HALF, NF), jnp.float32),
            jax.ShapeDtypeStruct((N_NODES, NF), jnp.float32),
        ],
    )(ea, w1t, b1, ew2d, w2t, b2, x, wlt)


def _edge_stage_b(ea, w1t, b1, ew2d, w2t, b2):
    return pl.pallas_call(
        _edge_body_b,
        grid=(N_TILES_HALF,),
        in_specs=[
            pl.BlockSpec((NRBF, EDGE_TILE), lambda i: (0, i + N_TILES_HALF)),
            pl.BlockSpec((NF, NRBF), lambda i: (0, 0)),
            pl.BlockSpec((1, NF), lambda i: (0, 0)),
            pl.BlockSpec((EDGE_TILE // 128, 128),
                         lambda i: (i + N_TILES_HALF, 0)),
            pl.BlockSpec((NF, NF), lambda i: (0, 0)),
            pl.BlockSpec((1, NF), lambda i: (0, 0)),
        ],
        out_specs=pl.BlockSpec((EDGE_TILE, NF), lambda i: (i, 0)),
        out_shape=jax.ShapeDtypeStruct((N_HALF, NF), jnp.float32),
    )(ea, w1t, b1, ew2d, w2t, b2)


def _make_sc_body(half):
    half_off = half * N_HALF

    def _sc_body(ei_hbm, z_hbm, xt_hbm, wf_hbm, y_hbm,
                 src0, src1, dst0, dst1, seg0, seg1,
                 rows0, rows1, wf0, wf1, zg0, zg1, zbuf_v, y_sh,
                 gsem0, gsem1, wsem0, wsem1, zsem0, zsem1):
        c = lax.axis_index("c")
        s = lax.axis_index("s")
        wid = c * NS + s
        ebase = half_off + wid * EH_PER_W
        wbase = wid * EH_PER_W
        srcs = [src0, src1]
        dsts = [dst0, dst1]
        segs = [seg0, seg1]
        rows = [rows0, rows1]
        wfs = [wf0, wf1]
        zgs = [zg0, zg1]
        gsems = [gsem0, gsem1]
        wsems = [wsem0, wsem1]
        zsems = [zsem0, zsem1]

        for kc in range(N_CHUNKS_H):
            pltpu.sync_copy(ei_hbm.at[0, pl.ds(ebase + kc * CHUNK, CHUNK)],
                            srcs[kc])
            pltpu.sync_copy(ei_hbm.at[1, pl.ds(ebase + kc * CHUNK, CHUNK)],
                            dsts[kc])

        gds, zds, wds = {}, {}, {}
        for kc in range(N_CHUNKS_H):
            gds[kc] = pltpu.make_async_copy(xt_hbm.at[srcs[kc]], rows[kc],
                                            gsems[kc])
            gds[kc].start()
            zds[kc] = pltpu.make_async_copy(z_hbm.at[srcs[kc]], zgs[kc],
                                            zsems[kc])
            zds[kc].start()
            wds[kc] = pltpu.make_async_copy(
                wf_hbm.at[pl.ds(wbase + kc * CHUNK, CHUNK)], wfs[kc],
                wsems[kc])
            wds[kc].start()

        zero16 = jnp.zeros((L,), jnp.float32)

        def zrow(i, carry):
            for j in range(NF // L):
                zbuf_v[i, pl.ds(j * L, L)] = zero16
            return carry

        lax.fori_loop(0, ROWS_PER_TILE // 5, zrow, 0)
        for r in range(5):
            pltpu.sync_copy(zbuf_v, y_sh.at[pl.ds(s * ROWS_PER_TILE +
                                                  r * (ROWS_PER_TILE // 5),
                                                  ROWS_PER_TILE // 5)])
        plsc.subcore_barrier()

        for kc in range(N_CHUNKS_H):
            zds[kc].wait()

            def segbody(i, carry):
                sl = pl.ds(i * L, L)
                segs[kc][sl] = zgs[kc][sl] * N_NODES + dsts[kc][sl]
                return carry

            lax.fori_loop(0, CHUNK // L, segbody, 0)
            gds[kc].wait()
            wds[kc].wait()

            def mulbody(i, carry):
                for j in range(NF // L):
                    sl = pl.ds(j * L, L)
                    rows[kc][i, sl] = rows[kc][i, sl] * wfs[kc][i, sl]
                return carry

            lax.fori_loop(0, CHUNK, mulbody, 0)
            pltpu.sync_copy(rows[kc], y_sh.at[segs[kc]], add=True)

        plsc.subcore_barrier()
        pltpu.sync_copy(y_sh.at[pl.ds(s * ROWS_PER_TILE, ROWS_PER_TILE)],
                        y_hbm.at[c, pl.ds(s * ROWS_PER_TILE, ROWS_PER_TILE)])

    return _sc_body


@functools.cache
def _sc_stage_fn(half):
    return functools.partial(
        pl.kernel,
        out_type=jax.ShapeDtypeStruct((NC, NSEG, NF), jnp.float32),
        mesh=plsc.VectorSubcoreMesh(core_axis_name="c", subcore_axis_name="s",
                                    num_cores=NC, num_subcores=NS),
        scratch_types=(
            [pltpu.VMEM((CHUNK,), jnp.int32)] * 4 +
            [pltpu.VMEM((CHUNK,), jnp.int32)] * 2 +
            [pltpu.VMEM((CHUNK, NF), jnp.float32)] * 2 +
            [pltpu.VMEM((CHUNK, NF), jnp.float32)] * 2 +
            [pltpu.VMEM((CHUNK,), jnp.int32)] * 2 +
            [pltpu.VMEM((ROWS_PER_TILE // 5, NF), jnp.float32),
             pltpu.VMEM_SHARED((NSEG, NF), jnp.float32)] +
            [pltpu.SemaphoreType.DMA] * 6
        ),
    )(_make_sc_body(half))


def _node_body(yp1_ref, yp2_ref, wqt_ref, wkt_ref, wvt_ref, wo1t_ref,
               wo2t_ref, out_ref):
    y = (yp1_ref[0] + yp1_ref[1]) + (yp2_ref[0] + yp2_ref[1])
    dnt = (((1,), (1,)), ((), ()))
    q = lax.dot_general(y, wqt_ref[...], dnt,
                        preferred_element_type=jnp.float32)
    k = lax.dot_general(y, wkt_ref[...], dnt,
                        preferred_element_type=jnp.float32)
    v = lax.dot_general(y, wvt_ref[...], dnt,
                        preferred_element_type=jnp.float32)
    dh = NF // NHEADS
    lane = lax.broadcasted_iota(jnp.int32, (NF, NHEADS), 0)
    head = lax.broadcasted_iota(jnp.int32, (NF, NHEADS), 1)
    em = (lane // dh == head).astype(jnp.float32)
    lane_t = lax.broadcasted_iota(jnp.int32, (NHEADS, NF), 1)
    head_t = lax.broadcasted_iota(jnp.int32, (NHEADS, NF), 0)
    em_t = (lane_t // dh == head_t).astype(jnp.float32)
    acc = jnp.zeros((N_NODES, NF), jnp.float32)
    for zq in range(NZ):
        qz = lax.slice(q, (zq * N_NODES, 0), ((zq + 1) * N_NODES, NF))
        for zk in range(NZ):
            ky = lax.slice(k, (zk * N_NODES, 0), ((zk + 1) * N_NODES, NF))
            vy = lax.slice(v, (zk * N_NODES, 0), ((zk + 1) * N_NODES, NF))
            sc = jnp.dot(qz * ky, em, preferred_element_type=jnp.float32)
            sc = sc * jax.nn.sigmoid(sc)
            a = jnp.dot(sc, em_t, preferred_element_type=jnp.float32)
            acc = acc + a * vy
    o = lax.dot_general(acc, wo1t_ref[...], dnt,
                        preferred_element_type=jnp.float32)
    o = lax.dot_general(o, wo2t_ref[...], dnt,
                        preferred_element_type=jnp.float32)
    out_ref[...] = o * jax.nn.sigmoid(o)


def _node_stage(yp1, yp2, wqt, wkt, wvt, wo1t, wo2t):
    return pl.pallas_call(
        _node_body,
        out_shape=jax.ShapeDtypeStruct((N_NODES, NF), jnp.float32),
    )(yp1, yp2, wqt, wkt, wvt, wo1t, wo2t)


def kernel(x, z, edge_index, edge_weight, edge_attr,
           W_lin1, W_f1, b_f1, W_f2, b_f2,
           Wq, bq, Wk, bk, Wv, bv, Wo1, bo1, Wo2, bo2):
    f32 = jnp.float32
    ea_t = edge_attr.astype(f32).T
    ew2 = edge_weight.astype(f32).reshape(N_EDGES // 128, 128)
    b1 = b_f1.reshape(1, NF)
    b2 = b_f2.reshape(1, NF)
    ei = edge_index.astype(jnp.int32)
    zi = z.astype(jnp.int32)

    wf_lo, xt = _edge_stage_a(ea_t, W_f1.astype(f32), b1, ew2,
                              W_f2.astype(f32), b2, x.astype(f32),
                              W_lin1.astype(f32))
    yp1 = _sc_stage_fn(0)(ei, zi, xt, wf_lo)
    wf_hi = _edge_stage_b(ea_t, W_f1.astype(f32), b1, ew2,
                          W_f2.astype(f32), b2)
    yp2 = _sc_stage_fn(1)(ei, zi, xt, wf_hi)

    out = _node_stage(yp1, yp2, Wq.astype(f32), Wk.astype(f32),
                      Wv.astype(f32), Wo1.astype(f32), Wo2.astype(f32))
    return out


# speedup vs baseline: 7.0302x; 1.0010x over previous
import functools
import numpy as np
import jax
import jax.numpy as jnp
from jax import lax
from jax.experimental import pallas as pl
from jax.experimental.pallas import tpu as pltpu
from jax.experimental.pallas import tpu_sc as plsc

N_NODES, N_EDGES, NF, NRBF, NHEADS = 512, 16384, 128, 50, 8
CUTOFF_UPPER = 5.0
NZ = 5
NSEG = NZ * N_NODES

NC, NS, L = 2, 16, 16
NW = NC * NS
N_HALF = N_EDGES // 2
EH_PER_W = N_HALF // NW
CHUNK = 128
N_CHUNKS_H = EH_PER_W // CHUNK
ROWS_PER_TILE = NSEG // NS

EDGE_TILE = 2048
N_TILES_HALF = N_HALF // EDGE_TILE


def _filter_block(ea_t, w1, b1, ew2, w2, b2):
    h = lax.dot_general(ea_t, w1, (((0,), (1,)), ((), ())),
                        preferred_element_type=jnp.float32)
    h = h + b1
    h = h * jax.nn.sigmoid(h)
    wf = lax.dot_general(h, w2, (((1,), (1,)), ((), ())),
                         preferred_element_type=jnp.float32)
    wf = wf + b2
    c = 0.5 * (jnp.cos(ew2 * (np.pi / CUTOFF_UPPER)) + 1.0)
    c = c * (ew2 < CUTOFF_UPPER).astype(jnp.float32)
    ct = c.T
    scaled = []
    for j in range(ct.shape[1]):
        col = lax.slice(ct, (0, j), (128, j + 1))
        slab = lax.slice(wf, (j * 128, 0), ((j + 1) * 128, NF))
        scaled.append(slab * col)
    return jnp.concatenate(scaled, axis=0)


def _edge_body_a(ea_ref, w1t_ref, b1_ref, ew_ref, w2t_ref, b2_ref, x_ref,
                 wlt_ref, wf_ref, xt_ref):
    wf_ref[...] = _filter_block(ea_ref[...], w1t_ref[...], b1_ref[...],
                                ew_ref[...], w2t_ref[...], b2_ref[...])

    @pl.when(pl.program_id(0) == 0)
    def _():
        xt_ref[...] = lax.dot_general(x_ref[...], wlt_ref[...],
                                      (((1,), (1,)), ((), ())),
                                      preferred_element_type=jnp.float32)


def _edge_body_b(ea_ref, w1t_ref, b1_ref, ew_ref, w2t_ref, b2_ref, wf_ref):
    wf_ref[...] = _filter_block(ea_ref[...], w1t_ref[...], b1_ref[...],
                                ew_ref[...], w2t_ref[...], b2_ref[...])


def _edge_stage_a(ea, w1t, b1, ew2d, w2t, b2, x, wlt):
    return pl.pallas_call(
        _edge_body_a,
        grid=(N_TILES_HALF,),
        in_specs=[
            pl.BlockSpec((NRBF, EDGE_TILE), lambda i: (0, i)),
            pl.BlockSpec((NF, NRBF), lambda i: (0, 0)),
            pl.BlockSpec((1, NF), lambda i: (0, 0)),
            pl.BlockSpec((EDGE_TILE // 128, 128), lambda i: (i, 0)),
            pl.BlockSpec((NF, NF), lambda i: (0, 0)),
            pl.BlockSpec((1, NF), lambda i: (0, 0)),
            pl.BlockSpec((N_NODES, NF), lambda i: (0, 0)),
            pl.BlockSpec((NF, NF), lambda i: (0, 0)),
        ],
        out_specs=[
            pl.BlockSpec((EDGE_TILE, NF), lambda i: (i, 0)),
            pl.BlockSpec((N_NODES, NF), lambda i: (0, 0)),
        ],
        out_shape=[
            jax.ShapeDtypeStruct((N_HALF, NF), jnp.float32),
            jax.ShapeDtypeStruct((N_NODES, NF), jnp.float32),
        ],
    )(ea, w1t, b1, ew2d, w2t, b2, x, wlt)


def _edge_stage_b(ea, w1t, b1, ew2d, w2t, b2):
    return pl.pallas_call(
        _edge_body_b,
        grid=(N_TILES_HALF,),
        in_specs=[
            pl.BlockSpec((NRBF, EDGE_TILE), lambda i: (0, i + N_TILES_HALF)),
            pl.BlockSpec((NF, NRBF), lambda i: (0, 0)),
            pl.BlockSpec((1, NF), lambda i: (0, 0)),
            pl.BlockSpec((EDGE_TILE // 128, 128),
                         lambda i: (i + N_TILES_HALF, 0)),
            pl.BlockSpec((NF, NF), lambda i: (0, 0)),
            pl.BlockSpec((1, NF), lambda i: (0, 0)),
        ],
        out_specs=pl.BlockSpec((EDGE_TILE, NF), lambda i: (i, 0)),
        out_shape=jax.ShapeDtypeStruct((N_HALF, NF), jnp.float32),
    )(ea, w1t, b1, ew2d, w2t, b2)


def _make_sc_body(half):
    half_off = half * N_HALF

    def _sc_body(ei_hbm, z_hbm, xt_hbm, wf_hbm, y_hbm,
                 src0, src1, dst0, dst1, seg0, seg1,
                 rows0, rows1, wf0, wf1, zg0, zg1, zbuf_v, y_sh,
                 gsem0, gsem1, wsem0, wsem1, zsem0, zsem1,
                 isem0, isem1, isem2, isem3,
                 csem0, csem1, csem2, csem3, csem4):
        c = lax.axis_index("c")
        s = lax.axis_index("s")
        wid = c * NS + s
        ebase = half_off + wid * EH_PER_W
        wbase = wid * EH_PER_W
        srcs = [src0, src1]
        dsts = [dst0, dst1]
        segs = [seg0, seg1]
        rows = [rows0, rows1]
        wfs = [wf0, wf1]
        zgs = [zg0, zg1]
        gsems = [gsem0, gsem1]
        wsems = [wsem0, wsem1]
        zsems = [zsem0, zsem1]
        isems = [isem0, isem1, isem2, isem3]
        csems = [csem0, csem1, csem2, csem3, csem4]

        ids = {}
        for kc in range(N_CHUNKS_H):
            ids[(0, kc)] = pltpu.make_async_copy(
                ei_hbm.at[0, pl.ds(ebase + kc * CHUNK, CHUNK)], srcs[kc],
                isems[kc])
            ids[(0, kc)].start()
            ids[(1, kc)] = pltpu.make_async_copy(
                ei_hbm.at[1, pl.ds(ebase + kc * CHUNK, CHUNK)], dsts[kc],
                isems[N_CHUNKS_H + kc])
            ids[(1, kc)].start()

        gds, zds, wds = {}, {}, {}
        for kc in range(N_CHUNKS_H):
            wds[kc] = pltpu.make_async_copy(
                wf_hbm.at[pl.ds(wbase + kc * CHUNK, CHUNK)], wfs[kc],
                wsems[kc])
            wds[kc].start()
        for kc in range(N_CHUNKS_H):
            ids[(0, kc)].wait()
            gds[kc] = pltpu.make_async_copy(xt_hbm.at[srcs[kc]], rows[kc],
                                            gsems[kc])
            gds[kc].start()
            zds[kc] = pltpu.make_async_copy(z_hbm.at[srcs[kc]], zgs[kc],
                                            zsems[kc])
            zds[kc].start()

        zero16 = jnp.zeros((L,), jnp.float32)

        def zrow(i, carry):
            for j in range(NF // L):
                zbuf_v[i, pl.ds(j * L, L)] = zero16
            return carry

        lax.fori_loop(0, ROWS_PER_TILE // 5, zrow, 0)
        zcs = []
        for r in range(5):
            zc = pltpu.make_async_copy(
                zbuf_v, y_sh.at[pl.ds(s * ROWS_PER_TILE +
                                      r * (ROWS_PER_TILE // 5),
                                      ROWS_PER_TILE // 5)], csems[r])
            zc.start()
            zcs.append(zc)
        for kc in range(N_CHUNKS_H):
            ids[(1, kc)].wait()
        for zc in zcs:
            zc.wait()
        plsc.subcore_barrier()

        for kc in range(N_CHUNKS_H):
            zds[kc].wait()

            def segbody(i, carry):
                sl = pl.ds(i * L, L)
                segs[kc][sl] = zgs[kc][sl] * N_NODES + dsts[kc][sl]
                return carry

            lax.fori_loop(0, CHUNK // L, segbody, 0)
            gds[kc].wait()
            wds[kc].wait()

            def mulbody(i, carry):
                for j in range(NF // L):
                    sl = pl.ds(j * L, L)
                    rows[kc][i, sl] = rows[kc][i, sl] * wfs[kc][i, sl]
                return carry

            lax.fori_loop(0, CHUNK, mulbody, 0)
            pltpu.sync_copy(rows[kc], y_sh.at[segs[kc]], add=True)

        plsc.subcore_barrier()
        pltpu.sync_copy(y_sh.at[pl.ds(s * ROWS_PER_TILE, ROWS_PER_TILE)],
                        y_hbm.at[c, pl.ds(s * ROWS_PER_TILE, ROWS_PER_TILE)])

    return _sc_body


@functools.cache
def _sc_stage_fn(half):
    return functools.partial(
        pl.kernel,
        out_type=jax.ShapeDtypeStruct((NC, NSEG, NF), jnp.float32),
        mesh=plsc.VectorSubcoreMesh(core_axis_name="c", subcore_axis_name="s",
                                    num_cores=NC, num_subcores=NS),
        scratch_types=(
            [pltpu.VMEM((CHUNK,), jnp.int32)] * 4 +
            [pltpu.VMEM((CHUNK,), jnp.int32)] * 2 +
            [pltpu.VMEM((CHUNK, NF), jnp.float32)] * 2 +
            [pltpu.VMEM((CHUNK, NF), jnp.float32)] * 2 +
            [pltpu.VMEM((CHUNK,), jnp.int32)] * 2 +
            [pltpu.VMEM((ROWS_PER_TILE // 5, NF), jnp.float32),
             pltpu.VMEM_SHARED((NSEG, NF), jnp.float32)] +
            [pltpu.SemaphoreType.DMA] * 15
        ),
    )(_make_sc_body(half))


def _node_body(yp1_ref, yp2_ref, wqt_ref, wkt_ref, wvt_ref, wo1t_ref,
               wo2t_ref, out_ref):
    y = (yp1_ref[0] + yp1_ref[1]) + (yp2_ref[0] + yp2_ref[1])
    dnt = (((1,), (1,)), ((), ()))
    q = lax.dot_general(y, wqt_ref[...], dnt,
                        preferred_element_type=jnp.float32)
    k = lax.dot_general(y, wkt_ref[...], dnt,
                        preferred_element_type=jnp.float32)
    v = lax.dot_general(y, wvt_ref[...], dnt,
                        preferred_element_type=jnp.float32)
    dh = NF // NHEADS
    lane = lax.broadcasted_iota(jnp.int32, (NF, NHEADS), 0)
    head = lax.broadcasted_iota(jnp.int32, (NF, NHEADS), 1)
    em = (lane // dh == head).astype(jnp.float32)
    lane_t = lax.broadcasted_iota(jnp.int32, (NHEADS, NF), 1)
    head_t = lax.broadcasted_iota(jnp.int32, (NHEADS, NF), 0)
    em_t = (lane_t // dh == head_t).astype(jnp.float32)
    acc = jnp.zeros((N_NODES, NF), jnp.float32)
    for zq in range(NZ):
        qz = lax.slice(q, (zq * N_NODES, 0), ((zq + 1) * N_NODES, NF))
        for zk in range(NZ):
            ky = lax.slice(k, (zk * N_NODES, 0), ((zk + 1) * N_NODES, NF))
            vy = lax.slice(v, (zk * N_NODES, 0), ((zk + 1) * N_NODES, NF))
            sc = jnp.dot(qz * ky, em, preferred_element_type=jnp.float32)
            sc = sc * jax.nn.sigmoid(sc)
            a = jnp.dot(sc, em_t, preferred_element_type=jnp.float32)
            acc = acc + a * vy
    o = lax.dot_general(acc, wo1t_ref[...], dnt,
                        preferred_element_type=jnp.float32)
    o = lax.dot_general(o, wo2t_ref[...], dnt,
                        preferred_element_type=jnp.float32)
    out_ref[...] = o * jax.nn.sigmoid(o)


def _node_stage(yp1, yp2, wqt, wkt, wvt, wo1t, wo2t):
    return pl.pallas_call(
        _node_body,
        out_shape=jax.ShapeDtypeStruct((N_NODES, NF), jnp.float32),
    )(yp1, yp2, wqt, wkt, wvt, wo1t, wo2t)


def kernel(x, z, edge_index, edge_weight, edge_attr,
           W_lin1, W_f1, b_f1, W_f2, b_f2,
           Wq, bq, Wk, bk, Wv, bv, Wo1, bo1, Wo2, bo2):
    f32 = jnp.float32
    ea_t = edge_attr.astype(f32).T
    ew2 = edge_weight.astype(f32).reshape(N_EDGES // 128, 128)
    b1 = b_f1.reshape(1, NF)
    b2 = b_f2.reshape(1, NF)
    ei = edge_index.astype(jnp.int32)
    zi = z.astype(jnp.int32)

    wf_lo, xt = _edge_stage_a(ea_t, W_f1.astype(f32), b1, ew2,
                              W_f2.astype(f32), b2, x.astype(f32),
                              W_lin1.astype(f32))
    yp1 = _sc_stage_fn(0)(ei, zi, xt, wf_lo)
    wf_hi = _edge_stage_b(ea_t, W_f1.astype(f32), b1, ew2,
                          W_f2.astype(f32), b2)
    yp2 = _sc_stage_fn(1)(ei, zi, xt, wf_hi)

    out = _node_stage(yp1, yp2, Wq.astype(f32), Wk.astype(f32),
                      Wv.astype(f32), Wo1.astype(f32), Wo2.astype(f32))
    return out


# speedup vs baseline: 7.9515x; 1.1311x over previous
import functools
import numpy as np
import jax
import jax.numpy as jnp
from jax import lax
from jax.experimental import pallas as pl
from jax.experimental.pallas import tpu as pltpu
from jax.experimental.pallas import tpu_sc as plsc

N_NODES, N_EDGES, NF, NRBF, NHEADS = 512, 16384, 128, 50, 8
CUTOFF_UPPER = 5.0
NZ = 5
NSEG = NZ * N_NODES

NC, NS, L = 2, 16, 16
NW = NC * NS
N_HALF = N_EDGES // 2
E_PER_W = N_EDGES // NW
CHUNK = 128
N_CHUNKS = 4
ROWS_PER_TILE = NSEG // NS

EDGE_TILE = 2048
N_TILES_HALF = N_HALF // EDGE_TILE
N_TILES_ALL = N_EDGES // EDGE_TILE


def _filter_block(ea_t, w1, b1, ew2, w2, b2):
    h = lax.dot_general(ea_t, w1, (((0,), (1,)), ((), ())),
                        preferred_element_type=jnp.float32)
    h = h + b1
    h = h * jax.nn.sigmoid(h)
    wf = lax.dot_general(h, w2, (((1,), (1,)), ((), ())),
                         preferred_element_type=jnp.float32)
    wf = wf + b2
    c = 0.5 * (jnp.cos(ew2 * (np.pi / CUTOFF_UPPER)) + 1.0)
    c = c * (ew2 < CUTOFF_UPPER).astype(jnp.float32)
    ct = c.T
    scaled = []
    for j in range(ct.shape[1]):
        col = lax.slice(ct, (0, j), (128, j + 1))
        slab = lax.slice(wf, (j * 128, 0), ((j + 1) * 128, NF))
        scaled.append(slab * col)
    return jnp.concatenate(scaled, axis=0)


def _edge_body_a(ea_ref, w1t_ref, b1_ref, ew_ref, w2t_ref, b2_ref, x_ref,
                 wlt_ref, wf_ref, xt_ref):
    wf_ref[...] = _filter_block(ea_ref[...], w1t_ref[...], b1_ref[...],
                                ew_ref[...], w2t_ref[...], b2_ref[...])

    @pl.when(pl.program_id(0) == 0)
    def _():
        xt_ref[...] = lax.dot_general(x_ref[...], wlt_ref[...],
                                      (((1,), (1,)), ((), ())),
                                      preferred_element_type=jnp.float32)


def _edge_body_b(ea_ref, w1t_ref, b1_ref, ew_ref, w2t_ref, b2_ref, wf_ref):
    wf_ref[...] = _filter_block(ea_ref[...], w1t_ref[...], b1_ref[...],
                                ew_ref[...], w2t_ref[...], b2_ref[...])


def _edge_stage_a(ea, w1t, b1, ew2d, w2t, b2, x, wlt):
    return pl.pallas_call(
        _edge_body_a,
        grid=(N_TILES_ALL,),
        in_specs=[
            pl.BlockSpec((NRBF, EDGE_TILE), lambda i: (0, i)),
            pl.BlockSpec((NF, NRBF), lambda i: (0, 0)),
            pl.BlockSpec((1, NF), lambda i: (0, 0)),
            pl.BlockSpec((EDGE_TILE // 128, 128), lambda i: (i, 0)),
            pl.BlockSpec((NF, NF), lambda i: (0, 0)),
            pl.BlockSpec((1, NF), lambda i: (0, 0)),
            pl.BlockSpec((N_NODES, NF), lambda i: (0, 0)),
            pl.BlockSpec((NF, NF), lambda i: (0, 0)),
        ],
        out_specs=[
            pl.BlockSpec((EDGE_TILE, NF), lambda i: (i, 0)),
            pl.BlockSpec((N_NODES, NF), lambda i: (0, 0)),
        ],
        out_shape=[
            jax.ShapeDtypeStruct((N_EDGES, NF), jnp.float32),
            jax.ShapeDtypeStruct((N_NODES, NF), jnp.float32),
        ],
    )(ea, w1t, b1, ew2d, w2t, b2, x, wlt)


def _edge_stage_b(ea, w1t, b1, ew2d, w2t, b2):
    return pl.pallas_call(
        _edge_body_b,
        grid=(N_TILES_HALF,),
        in_specs=[
            pl.BlockSpec((NRBF, EDGE_TILE), lambda i: (0, i + N_TILES_HALF)),
            pl.BlockSpec((NF, NRBF), lambda i: (0, 0)),
            pl.BlockSpec((1, NF), lambda i: (0, 0)),
            pl.BlockSpec((EDGE_TILE // 128, 128),
                         lambda i: (i + N_TILES_HALF, 0)),
            pl.BlockSpec((NF, NF), lambda i: (0, 0)),
            pl.BlockSpec((1, NF), lambda i: (0, 0)),
        ],
        out_specs=pl.BlockSpec((EDGE_TILE, NF), lambda i: (i, 0)),
        out_shape=jax.ShapeDtypeStruct((N_HALF, NF), jnp.float32),
    )(ea, w1t, b1, ew2d, w2t, b2)


def _sc_body(ei_hbm, z_hbm, xt_hbm, wf_hbm, y_hbm,
             src0, src1, src2, src3, dst0, dst1, dst2, dst3,
             seg0, seg1, rows0, rows1, wf0, wf1, zg0, zg1, zbuf_v, y_sh,
             gsem0, gsem1, wsem0, wsem1, zsem0, zsem1,
             isem0, isem1, isem2, isem3, isem4, isem5, isem6, isem7,
             csem0, csem1, csem2, csem3, csem4):
    c = lax.axis_index("c")
    s = lax.axis_index("s")
    wid = c * NS + s
    ebase = wid * E_PER_W
    srcs = [src0, src1, src2, src3]
    dsts = [dst0, dst1, dst2, dst3]
    segs = [seg0, seg1]
    rows = [rows0, rows1]
    wfs = [wf0, wf1]
    zgs = [zg0, zg1]
    gsems = [gsem0, gsem1]
    wsems = [wsem0, wsem1]
    zsems = [zsem0, zsem1]
    isems = [isem0, isem1, isem2, isem3, isem4, isem5, isem6, isem7]
    csems = [csem0, csem1, csem2, csem3, csem4]

    ids = {}
    for kc in range(N_CHUNKS):
        ids[(0, kc)] = pltpu.make_async_copy(
            ei_hbm.at[0, pl.ds(ebase + kc * CHUNK, CHUNK)], srcs[kc],
            isems[kc])
        ids[(0, kc)].start()
        ids[(1, kc)] = pltpu.make_async_copy(
            ei_hbm.at[1, pl.ds(ebase + kc * CHUNK, CHUNK)], dsts[kc],
            isems[N_CHUNKS + kc])
        ids[(1, kc)].start()

    gds, zds, wds = {}, {}, {}

    def start_fetch(kc):
        gds[kc] = pltpu.make_async_copy(xt_hbm.at[srcs[kc]], rows[kc % 2],
                                        gsems[kc % 2])
        gds[kc].start()
        zds[kc] = pltpu.make_async_copy(z_hbm.at[srcs[kc]], zgs[kc % 2],
                                        zsems[kc % 2])
        zds[kc].start()
        wds[kc] = pltpu.make_async_copy(
            wf_hbm.at[pl.ds(ebase + kc * CHUNK, CHUNK)], wfs[kc % 2],
            wsems[kc % 2])
        wds[kc].start()

    for kc in range(2):
        ids[(0, kc)].wait()
        start_fetch(kc)

    zero16 = jnp.zeros((L,), jnp.float32)

    def zrow(i, carry):
        for j in range(NF // L):
            zbuf_v[i, pl.ds(j * L, L)] = zero16
        return carry

    lax.fori_loop(0, ROWS_PER_TILE // 5, zrow, 0)
    zcs = []
    for r in range(5):
        zc = pltpu.make_async_copy(
            zbuf_v, y_sh.at[pl.ds(s * ROWS_PER_TILE +
                                  r * (ROWS_PER_TILE // 5),
                                  ROWS_PER_TILE // 5)], csems[r])
        zc.start()
        zcs.append(zc)
    for kc in range(2, N_CHUNKS):
        ids[(0, kc)].wait()
    for kc in range(N_CHUNKS):
        ids[(1, kc)].wait()
    for zc in zcs:
        zc.wait()
    plsc.subcore_barrier()

    for kc in range(N_CHUNKS):
        cur2 = kc % 2
        zds[kc].wait()

        def segbody(i, carry):
            sl = pl.ds(i * L, L)
            segs[cur2][sl] = zgs[cur2][sl] * N_NODES + dsts[kc][sl]
            return carry

        lax.fori_loop(0, CHUNK // L, segbody, 0)
        gds[kc].wait()
        wds[kc].wait()

        def mulbody(i, carry):
            for j in range(NF // L):
                sl = pl.ds(j * L, L)
                rows[cur2][i, sl] = rows[cur2][i, sl] * wfs[cur2][i, sl]
            return carry

        lax.fori_loop(0, CHUNK, mulbody, 0)
        pltpu.sync_copy(rows[cur2], y_sh.at[segs[cur2]], add=True)
        if kc + 2 < N_CHUNKS:
            start_fetch(kc + 2)

    plsc.subcore_barrier()
    pltpu.sync_copy(y_sh.at[pl.ds(s * ROWS_PER_TILE, ROWS_PER_TILE)],
                    y_hbm.at[c, pl.ds(s * ROWS_PER_TILE, ROWS_PER_TILE)])


@functools.cache
def _sc_stage_fn():
    return functools.partial(
        pl.kernel,
        out_type=jax.ShapeDtypeStruct((NC, NSEG, NF), jnp.float32),
        mesh=plsc.VectorSubcoreMesh(core_axis_name="c", subcore_axis_name="s",
                                    num_cores=NC, num_subcores=NS),
        scratch_types=(
            [pltpu.VMEM((CHUNK,), jnp.int32)] * 8 +
            [pltpu.VMEM((CHUNK,), jnp.int32)] * 2 +
            [pltpu.VMEM((CHUNK, NF), jnp.float32)] * 2 +
            [pltpu.VMEM((CHUNK, NF), jnp.float32)] * 2 +
            [pltpu.VMEM((CHUNK,), jnp.int32)] * 2 +
            [pltpu.VMEM((ROWS_PER_TILE // 5, NF), jnp.float32),
             pltpu.VMEM_SHARED((NSEG, NF), jnp.float32)] +
            [pltpu.SemaphoreType.DMA] * 19
        ),
    )(_sc_body)


def _node_body(yp1_ref, wqt_ref, wkt_ref, wvt_ref, wo1t_ref,
               wo2t_ref, out_ref):
    y = yp1_ref[0] + yp1_ref[1]
    dnt = (((1,), (1,)), ((), ()))
    q = lax.dot_general(y, wqt_ref[...], dnt,
                        preferred_element_type=jnp.float32)
    k = lax.dot_general(y, wkt_ref[...], dnt,
                        preferred_element_type=jnp.float32)
    v = lax.dot_general(y, wvt_ref[...], dnt,
                        preferred_element_type=jnp.float32)
    dh = NF // NHEADS
    lane = lax.broadcasted_iota(jnp.int32, (NF, NHEADS), 0)
    head = lax.broadcasted_iota(jnp.int32, (NF, NHEADS), 1)
    em = (lane // dh == head).astype(jnp.float32)
    lane_t = lax.broadcasted_iota(jnp.int32, (NHEADS, NF), 1)
    head_t = lax.broadcasted_iota(jnp.int32, (NHEADS, NF), 0)
    em_t = (lane_t // dh == head_t).astype(jnp.float32)
    acc = jnp.zeros((N_NODES, NF), jnp.float32)
    for zq in range(NZ):
        qz = lax.slice(q, (zq * N_NODES, 0), ((zq + 1) * N_NODES, NF))
        for zk in range(NZ):
            ky = lax.slice(k, (zk * N_NODES, 0), ((zk + 1) * N_NODES, NF))
            vy = lax.slice(v, (zk * N_NODES, 0), ((zk + 1) * N_NODES, NF))
            sc = jnp.dot(qz * ky, em, preferred_element_type=jnp.float32)
            sc = sc * jax.nn.sigmoid(sc)
            a = jnp.dot(sc, em_t, preferred_element_type=jnp.float32)
            acc = acc + a * vy
    o = lax.dot_general(acc, wo1t_ref[...], dnt,
                        preferred_element_type=jnp.float32)
    o = lax.dot_general(o, wo2t_ref[...], dnt,
                        preferred_element_type=jnp.float32)
    out_ref[...] = o * jax.nn.sigmoid(o)


def _node_stage(yp1, wqt, wkt, wvt, wo1t, wo2t):
    return pl.pallas_call(
        _node_body,
        out_shape=jax.ShapeDtypeStruct((N_NODES, NF), jnp.float32),
    )(yp1, wqt, wkt, wvt, wo1t, wo2t)


def kernel(x, z, edge_index, edge_weight, edge_attr,
           W_lin1, W_f1, b_f1, W_f2, b_f2,
           Wq, bq, Wk, bk, Wv, bv, Wo1, bo1, Wo2, bo2):
    f32 = jnp.float32
    ea_t = edge_attr.astype(f32).T
    ew2 = edge_weight.astype(f32).reshape(N_EDGES // 128, 128)
    b1 = b_f1.reshape(1, NF)
    b2 = b_f2.reshape(1, NF)
    ei = edge_index.astype(jnp.int32)
    zi = z.astype(jnp.int32)

    wf, xt = _edge_stage_a(ea_t, W_f1.astype(f32), b1, ew2,
                           W_f2.astype(f32), b2, x.astype(f32),
                           W_lin1.astype(f32))
    yp1 = _sc_stage_fn()(ei, zi, xt, wf)

    out = _node_stage(yp1, Wq.astype(f32), Wk.astype(f32),
                      Wv.astype(f32), Wo1.astype(f32), Wo2.astype(f32))
    return out


# speedup vs baseline: 8.0207x; 1.0087x over previous
import functools
import numpy as np
import jax
import jax.numpy as jnp
from jax import lax
from jax.experimental import pallas as pl
from jax.experimental.pallas import tpu as pltpu
from jax.experimental.pallas import tpu_sc as plsc

N_NODES, N_EDGES, NF, NRBF, NHEADS = 512, 16384, 128, 50, 8
CUTOFF_UPPER = 5.0
NZ = 5
NSEG = NZ * N_NODES

NC, NS, L = 2, 16, 16
NW = NC * NS
E_PER_W = N_EDGES // NW
CHUNK = 128
N_CHUNKS = 4
ROWS_PER_TILE = NSEG // NS

EDGE_TILE = 2048
N_TILES_ALL = N_EDGES // EDGE_TILE


def _filter_block(ea_t, w1, b1, ew2, w2, b2):
    h = lax.dot_general(ea_t, w1, (((0,), (1,)), ((), ())),
                        preferred_element_type=jnp.float32)
    h = h + b1
    h = h * jax.nn.sigmoid(h)
    wf = lax.dot_general(h, w2, (((1,), (1,)), ((), ())),
                         preferred_element_type=jnp.float32)
    wf = wf + b2
    c = 0.5 * (jnp.cos(ew2 * (np.pi / CUTOFF_UPPER)) + 1.0)
    c = c * (ew2 < CUTOFF_UPPER).astype(jnp.float32)
    ct = c.T
    scaled = []
    for j in range(ct.shape[1]):
        col = lax.slice(ct, (0, j), (128, j + 1))
        slab = lax.slice(wf, (j * 128, 0), ((j + 1) * 128, NF))
        scaled.append(slab * col)
    return jnp.concatenate(scaled, axis=0)


def _edge_body_a(ea_ref, w1t_ref, b1_ref, ew_ref, w2t_ref, b2_ref, x_ref,
                 wlt_ref, wf_ref, xt_ref):
    wf_ref[...] = _filter_block(ea_ref[...], w1t_ref[...], b1_ref[...],
                                ew_ref[...], w2t_ref[...], b2_ref[...])

    @pl.when(pl.program_id(0) == 0)
    def _():
        xt_ref[...] = lax.dot_general(x_ref[...], wlt_ref[...],
                                      (((1,), (1,)), ((), ())),
                                      preferred_element_type=jnp.float32)


def _edge_stage_a(ea, w1t, b1, ew2d, w2t, b2, x, wlt):
    return pl.pallas_call(
        _edge_body_a,
        grid=(N_TILES_ALL,),
        in_specs=[
            pl.BlockSpec((NRBF, EDGE_TILE), lambda i: (0, i)),
            pl.BlockSpec((NF, NRBF), lambda i: (0, 0)),
            pl.BlockSpec((1, NF), lambda i: (0, 0)),
            pl.BlockSpec((EDGE_TILE // 128, 128), lambda i: (i, 0)),
            pl.BlockSpec((NF, NF), lambda i: (0, 0)),
            pl.BlockSpec((1, NF), lambda i: (0, 0)),
            pl.BlockSpec((N_NODES, NF), lambda i: (0, 0)),
            pl.BlockSpec((NF, NF), lambda i: (0, 0)),
        ],
        out_specs=[
            pl.BlockSpec((EDGE_TILE, NF), lambda i: (i, 0)),
            pl.BlockSpec((N_NODES, NF), lambda i: (0, 0)),
        ],
        out_shape=[
            jax.ShapeDtypeStruct((N_EDGES, NF), jnp.float32),
            jax.ShapeDtypeStruct((N_NODES, NF), jnp.float32),
        ],
    )(ea, w1t, b1, ew2d, w2t, b2, x, wlt)


def _sc_body(ei_hbm, z_hbm, xt_hbm, wf_hbm, y_hbm,
             src0, src1, src2, src3, dst0, dst1, dst2, dst3,
             seg0, seg1, rows0, rows1, wf0, wf1, zg0, zg1, zbuf_v, y_sh,
             gsem0, gsem1, wsem0, wsem1, zsem0, zsem1,
             isem0, isem1, isem2, isem3, isem4, isem5, isem6, isem7,
             csem0, csem1, csem2, csem3, csem4):
    c = lax.axis_index("c")
    s = lax.axis_index("s")
    wid = c * NS + s
    ebase = wid * E_PER_W
    srcs = [src0, src1, src2, src3]
    dsts = [dst0, dst1, dst2, dst3]
    segs = [seg0, seg1]
    rows = [rows0, rows1]
    wfs = [wf0, wf1]
    zgs = [zg0, zg1]
    gsems = [gsem0, gsem1]
    wsems = [wsem0, wsem1]
    zsems = [zsem0, zsem1]
    isems = [isem0, isem1, isem2, isem3, isem4, isem5, isem6, isem7]
    csems = [csem0, csem1, csem2, csem3, csem4]

    ids = {}
    for kc in range(N_CHUNKS):
        ids[(0, kc)] = pltpu.make_async_copy(
            ei_hbm.at[0, pl.ds(ebase + kc * CHUNK, CHUNK)], srcs[kc],
            isems[kc])
        ids[(0, kc)].start()
        ids[(1, kc)] = pltpu.make_async_copy(
            ei_hbm.at[1, pl.ds(ebase + kc * CHUNK, CHUNK)], dsts[kc],
            isems[N_CHUNKS + kc])
        ids[(1, kc)].start()

    gds, zds, wds = {}, {}, {}

    def start_fetch(kc):
        gds[kc] = pltpu.make_async_copy(xt_hbm.at[srcs[kc]], rows[kc % 2],
                                        gsems[kc % 2])
        gds[kc].start()
        zds[kc] = pltpu.make_async_copy(z_hbm.at[srcs[kc]], zgs[kc % 2],
                                        zsems[kc % 2])
        zds[kc].start()
        wds[kc] = pltpu.make_async_copy(
            wf_hbm.at[pl.ds(ebase + kc * CHUNK, CHUNK)], wfs[kc % 2],
            wsems[kc % 2])
        wds[kc].start()

    for kc in range(2):
        ids[(0, kc)].wait()
        start_fetch(kc)

    zero16 = jnp.zeros((L,), jnp.float32)

    def zrow(i, carry):
        for j in range(NF // L):
            zbuf_v[i, pl.ds(j * L, L)] = zero16
        return carry

    lax.fori_loop(0, ROWS_PER_TILE // 5, zrow, 0)
    zcs = []
    for r in range(5):
        zc = pltpu.make_async_copy(
            zbuf_v, y_sh.at[pl.ds(s * ROWS_PER_TILE +
                                  r * (ROWS_PER_TILE // 5),
                                  ROWS_PER_TILE // 5)], csems[r])
        zc.start()
        zcs.append(zc)
    for kc in range(2, N_CHUNKS):
        ids[(0, kc)].wait()
    for kc in range(N_CHUNKS):
        ids[(1, kc)].wait()
    for zc in zcs:
        zc.wait()
    plsc.subcore_barrier()

    for kc in range(N_CHUNKS):
        cur2 = kc % 2
        zds[kc].wait()

        def segbody(i, carry):
            sl = pl.ds(i * L, L)
            segs[cur2][sl] = zgs[cur2][sl] * N_NODES + dsts[kc][sl]
            return carry

        lax.fori_loop(0, CHUNK // L, segbody, 0)
        gds[kc].wait()
        wds[kc].wait()

        def mulbody(i, carry):
            for j in range(NF // L):
                sl = pl.ds(j * L, L)
                rows[cur2][i, sl] = rows[cur2][i, sl] * wfs[cur2][i, sl]
            return carry

        lax.fori_loop(0, CHUNK, mulbody, 0)
        pltpu.sync_copy(rows[cur2], y_sh.at[segs[cur2]], add=True)
        if kc + 2 < N_CHUNKS:
            start_fetch(kc + 2)

    plsc.subcore_barrier()
    pltpu.sync_copy(y_sh.at[pl.ds(s * ROWS_PER_TILE, ROWS_PER_TILE)],
                    y_hbm.at[c, pl.ds(s * ROWS_PER_TILE, ROWS_PER_TILE)])


@functools.cache
def _sc_stage_fn():
    return functools.partial(
        pl.kernel,
        out_type=jax.ShapeDtypeStruct((NC, NSEG, NF), jnp.float32),
        mesh=plsc.VectorSubcoreMesh(core_axis_name="c", subcore_axis_name="s",
                                    num_cores=NC, num_subcores=NS),
        scratch_types=(
            [pltpu.VMEM((CHUNK,), jnp.int32)] * 8 +
            [pltpu.VMEM((CHUNK,), jnp.int32)] * 2 +
            [pltpu.VMEM((CHUNK, NF), jnp.float32)] * 2 +
            [pltpu.VMEM((CHUNK, NF), jnp.float32)] * 2 +
            [pltpu.VMEM((CHUNK,), jnp.int32)] * 2 +
            [pltpu.VMEM((ROWS_PER_TILE // 5, NF), jnp.float32),
             pltpu.VMEM_SHARED((NSEG, NF), jnp.float32)] +
            [pltpu.SemaphoreType.DMA] * 19
        ),
    )(_sc_body)


def _node_body(yp1_ref, wqt_ref, wkt_ref, wvt_ref, wo1t_ref,
               wo2t_ref, out_ref):
    y = yp1_ref[0] + yp1_ref[1]
    dnt = (((1,), (1,)), ((), ()))
    q = lax.dot_general(y, wqt_ref[...], dnt,
                        preferred_element_type=jnp.float32)
    k = lax.dot_general(y, wkt_ref[...], dnt,
                        preferred_element_type=jnp.float32)
    v = lax.dot_general(y, wvt_ref[...], dnt,
                        preferred_element_type=jnp.float32)
    dh = NF // NHEADS
    lane = lax.broadcasted_iota(jnp.int32, (NF, NHEADS), 0)
    head = lax.broadcasted_iota(jnp.int32, (NF, NHEADS), 1)
    em = (lane // dh == head).astype(jnp.float32)
    lane_t = lax.broadcasted_iota(jnp.int32, (NHEADS, NF), 1)
    head_t = lax.broadcasted_iota(jnp.int32, (NHEADS, NF), 0)
    em_t = (lane_t // dh == head_t).astype(jnp.float32)
    acc = jnp.zeros((N_NODES, NF), jnp.float32)
    for zq in range(NZ):
        qz = lax.slice(q, (zq * N_NODES, 0), ((zq + 1) * N_NODES, NF))
        for zk in range(NZ):
            ky = lax.slice(k, (zk * N_NODES, 0), ((zk + 1) * N_NODES, NF))
            vy = lax.slice(v, (zk * N_NODES, 0), ((zk + 1) * N_NODES, NF))
            sc = jnp.dot(qz * ky, em, preferred_element_type=jnp.float32)
            sc = sc * jax.nn.sigmoid(sc)
            a = jnp.dot(sc, em_t, preferred_element_type=jnp.float32)
            acc = acc + a * vy
    o = lax.dot_general(acc, wo1t_ref[...], dnt,
                        preferred_element_type=jnp.float32)
    o = lax.dot_general(o, wo2t_ref[...], dnt,
                        preferred_element_type=jnp.float32)
    out_ref[...] = o * jax.nn.sigmoid(o)


def _node_stage(yp1, wqt, wkt, wvt, wo1t, wo2t):
    return pl.pallas_call(
        _node_body,
        out_shape=jax.ShapeDtypeStruct((N_NODES, NF), jnp.float32),
    )(yp1, wqt, wkt, wvt, wo1t, wo2t)


def kernel(x, z, edge_index, edge_weight, edge_attr,
           W_lin1, W_f1, b_f1, W_f2, b_f2,
           Wq, bq, Wk, bk, Wv, bv, Wo1, bo1, Wo2, bo2):
    f32 = jnp.float32
    ea_t = edge_attr.astype(f32).T
    ew2 = edge_weight.astype(f32).reshape(N_EDGES // 128, 128)
    b1 = b_f1.reshape(1, NF)
    b2 = b_f2.reshape(1, NF)
    ei = edge_index.astype(jnp.int32)
    zi = z.astype(jnp.int32)

    wf, xt = _edge_stage_a(ea_t, W_f1.astype(f32), b1, ew2,
                           W_f2.astype(f32), b2, x.astype(f32),
                           W_lin1.astype(f32))
    yp1 = _sc_stage_fn()(ei, zi, xt, wf)

    out = _node_stage(yp1, Wq.astype(f32), Wk.astype(f32),
                      Wv.astype(f32), Wo1.astype(f32), Wo2.astype(f32))
    return out
